# Initial kernel scaffold; baseline (speedup 1.0000x reference)
import jax
import jax.numpy as jnp
from jax.experimental import pallas as pl


def kernel(img_lr, lsb_weight):
    raise NotImplementedError("write your pallas kernel here")


# speedup vs baseline: 325.9426x; 325.9426x over previous
import functools

import jax
import jax.numpy as jnp
from jax import lax
from jax.experimental import pallas as pl
from jax.experimental.pallas import tpu as pltpu
from jax.experimental.pallas import tpu_sc as plsc

L = 16
UPSCALE = 2
H = 2048
W = 2048
WPAD = 2064
NTILES = 32
ROWS_PER_TILE = H // NTILES
CHUNK = 4
NCHUNKS = ROWS_PER_TILE // CHUNK
CBLKS = W // 16
OW = W * UPSCALE

_PERMS = ((0, 1, 2, 3), (2, 0, 3, 1), (3, 2, 1, 0), (1, 3, 0, 2))


def _build_table(lsb_weight):
    wt = 0.5 * jnp.trunc(lsb_weight.astype(jnp.float32))
    parts = []
    for t in range(8):
        k, r = (0, t) if t < 4 else (1, t - 4)
        parts.append(wt[k][:, jnp.array(_PERMS[r])])
    return jnp.stack(parts).reshape(-1)


def _sc_kernel(p_hbm, t_hbm, out_hbm, tbl, ibuf, obuf):
    nc = 2
    wid = lax.axis_index("s") * nc + lax.axis_index("c")
    row0 = wid * ROWS_PER_TILE
    pltpu.sync_copy(t_hbm, tbl)
    ii = lax.iota(jnp.int32, 16)
    ii2 = ii * 2

    def chunk_body(ck, _):
        r_in = row0 + ck * CHUNK
        def stage(d, __):
            pltpu.sync_copy(
                p_hbm.at[pl.ds((r_in + d) * WPAD, WPAD)],
                ibuf.at[pl.ds(d * WPAD, WPAD)],
            )
            return __
        lax.fori_loop(0, CHUNK + 2, stage, 0, unroll=True)

        def px_body(k, __):
            r = k >> 7
            cb = k & 127
            j0 = cb * 16
            off_u = r * WPAD + j0
            off_c = off_u + WPAD
            off_d = off_c + WPAD
            a = ibuf[pl.ds(off_c + 1, 16)]
            nb_r = ibuf[pl.ds(off_c + 2, 16)]
            nb_l = ibuf[pl.ds(off_c, 16)]
            nb_u = ibuf[pl.ds(off_u + 1, 16)]
            nb_d = ibuf[pl.ds(off_d + 1, 16)]
            nb_dr = ibuf[pl.ds(off_d + 2, 16)]
            nb_dl = ibuf[pl.ds(off_d, 16)]
            nb_ul = ibuf[pl.ds(off_u, 16)]
            nb_ur = ibuf[pl.ds(off_u + 2, 16)]
            a64 = a << 6
            acc = [None, None, None, None]
            for t, nb in enumerate(
                (nb_r, nb_d, nb_l, nb_u, nb_dr, nb_dl, nb_ul, nb_ur)
            ):
                base = a64 + (nb << 2) + (t * 1024)
                for c in range(4):
                    g = plsc.load_gather(tbl, [base + c])
                    acc[c] = g if acc[c] is None else acc[c] + g
            b00 = (r * 2) * OW + j0 * 2 + ii2
            b10 = b00 + OW
            plsc.store_scatter(obuf, [b00], acc[0])
            plsc.store_scatter(obuf, [b00 + 1], acc[1])
            plsc.store_scatter(obuf, [b10], acc[2])
            plsc.store_scatter(obuf, [b10 + 1], acc[3])
            return __

        lax.fori_loop(0, CHUNK * CBLKS, px_body, 0)
        pltpu.sync_copy(
            obuf, out_hbm.at[pl.ds(r_in * UPSCALE * OW, CHUNK * UPSCALE * OW)]
        )
        return _

    lax.fori_loop(0, NCHUNKS, chunk_body, 0)


@functools.partial(jax.jit, static_argnames=())
def kernel(img_lr, lsb_weight):
    img = img_lr.astype(jnp.int32)
    p = jnp.pad(img, 1, mode="reflect")
    p = jnp.pad(p, ((0, 0), (0, WPAD - (W + 2))))
    p_flat = p.reshape(-1)
    t_flat = _build_table(lsb_weight)

    mesh = plsc.VectorSubcoreMesh(core_axis_name="c", subcore_axis_name="s")
    run = functools.partial(
        pl.kernel,
        mesh=mesh,
        out_type=jax.ShapeDtypeStruct((UPSCALE * H * OW,), jnp.float32),
        scratch_types=[
            pltpu.VMEM((8 * 256 * 4,), jnp.float32),
            pltpu.VMEM(((CHUNK + 2) * WPAD,), jnp.int32),
            pltpu.VMEM((CHUNK * UPSCALE * OW,), jnp.float32),
        ],
        compiler_params=pltpu.CompilerParams(needs_layout_passes=False),
    )(_sc_kernel)
    out_flat = run(p_flat, t_flat)
    return out_flat.reshape(UPSCALE * H, UPSCALE * W)


# speedup vs baseline: 412.0241x; 1.2641x over previous
import functools

import jax
import jax.numpy as jnp
from jax import lax
from jax.experimental import pallas as pl
from jax.experimental.pallas import tpu as pltpu
from jax.experimental.pallas import tpu_sc as plsc

L = 16
UPSCALE = 2
H = 2048
W = 2048
WPAD = 2064
NTILES = 32
ROWS_PER_TILE = H // NTILES
CHUNK = 4
NCHUNKS = ROWS_PER_TILE // CHUNK
CBLKS = W // 16
OW = W * UPSCALE

_PERMS = ((0, 1, 2, 3), (2, 0, 3, 1), (3, 2, 1, 0), (1, 3, 0, 2))


def _build_table(lsb_weight):
    wt = 0.5 * jnp.trunc(lsb_weight.astype(jnp.float32))
    parts = []
    for t in range(8):
        k, r = (0, t) if t < 4 else (1, t - 4)
        parts.append(wt[k][:, jnp.array(_PERMS[r])])
    tt = jnp.stack(parts)
    u = lax.bitcast_convert_type(tt.astype(jnp.bfloat16), jnp.uint16).astype(
        jnp.uint32
    )
    pair01 = u[..., 0] | (u[..., 1] << 16)
    pair23 = u[..., 2] | (u[..., 3] << 16)
    packed = jnp.stack([pair01, pair23], axis=-1)
    return lax.bitcast_convert_type(packed, jnp.int32).reshape(-1)


def _sc_kernel(p_hbm, t_hbm, out_hbm, tbl, ibuf, obuf):
    nc = 2
    wid = lax.axis_index("s") * nc + lax.axis_index("c")
    row0 = wid * ROWS_PER_TILE
    pltpu.sync_copy(t_hbm, tbl)
    ii = lax.iota(jnp.int32, 16)
    ii2 = ii * 2

    def chunk_body(ck, _):
        r_in = row0 + ck * CHUNK
        def stage(d, __):
            pltpu.sync_copy(
                p_hbm.at[pl.ds((r_in + d) * WPAD, WPAD)],
                ibuf.at[pl.ds(d * WPAD, WPAD)],
            )
            return __
        lax.fori_loop(0, CHUNK + 2, stage, 0, unroll=True)

        def px_body(k, __):
            r = k >> 7
            cb = k & 127
            j0 = cb * 16
            off_u = r * WPAD + j0
            off_c = off_u + WPAD
            off_d = off_c + WPAD
            a = ibuf[pl.ds(off_c + 1, 16)]
            nb_r = ibuf[pl.ds(off_c + 2, 16)]
            nb_l = ibuf[pl.ds(off_c, 16)]
            nb_u = ibuf[pl.ds(off_u + 1, 16)]
            nb_d = ibuf[pl.ds(off_d + 1, 16)]
            nb_dr = ibuf[pl.ds(off_d + 2, 16)]
            nb_dl = ibuf[pl.ds(off_d, 16)]
            nb_ul = ibuf[pl.ds(off_u, 16)]
            nb_ur = ibuf[pl.ds(off_u + 2, 16)]
            a32 = a << 5
            acc = [None, None, None, None]
            for t, nb in enumerate(
                (nb_r, nb_d, nb_l, nb_u, nb_dr, nb_dl, nb_ul, nb_ur)
            ):
                base = a32 + (nb << 1) + (t * 512)
                g01 = plsc.load_gather(tbl, [base])
                g23 = plsc.load_gather(tbl, [base + 1])
                c0, c1 = plsc.unpack(
                    plsc.bitcast(g01, jnp.bfloat16),
                    format=plsc.PackFormat.INTERLEAVED,
                )
                c2, c3 = plsc.unpack(
                    plsc.bitcast(g23, jnp.bfloat16),
                    format=plsc.PackFormat.INTERLEAVED,
                )
                for c, g in enumerate((c0, c1, c2, c3)):
                    acc[c] = g if acc[c] is None else acc[c] + g
            b00 = (r * 2) * OW + j0 * 2 + ii2
            b10 = b00 + OW
            plsc.store_scatter(obuf, [b00], acc[0])
            plsc.store_scatter(obuf, [b00 + 1], acc[1])
            plsc.store_scatter(obuf, [b10], acc[2])
            plsc.store_scatter(obuf, [b10 + 1], acc[3])
            return __

        lax.fori_loop(0, CHUNK * CBLKS, px_body, 0)
        pltpu.sync_copy(
            obuf, out_hbm.at[pl.ds(r_in * UPSCALE * OW, CHUNK * UPSCALE * OW)]
        )
        return _

    lax.fori_loop(0, NCHUNKS, chunk_body, 0)


@functools.partial(jax.jit, static_argnames=())
def kernel(img_lr, lsb_weight):
    img = img_lr.astype(jnp.int32)
    p = jnp.pad(img, 1, mode="reflect")
    p = jnp.pad(p, ((0, 0), (0, WPAD - (W + 2))))
    p_flat = p.reshape(-1)
    t_flat = _build_table(lsb_weight)

    mesh = plsc.VectorSubcoreMesh(core_axis_name="c", subcore_axis_name="s")
    run = functools.partial(
        pl.kernel,
        mesh=mesh,
        out_type=jax.ShapeDtypeStruct((UPSCALE * H * OW,), jnp.float32),
        scratch_types=[
            pltpu.VMEM((8 * 256 * 2,), jnp.int32),
            pltpu.VMEM(((CHUNK + 2) * WPAD,), jnp.int32),
            pltpu.VMEM((CHUNK * UPSCALE * OW,), jnp.float32),
        ],
        compiler_params=pltpu.CompilerParams(needs_layout_passes=False),
    )(_sc_kernel)
    out_flat = run(p_flat, t_flat)
    return out_flat.reshape(UPSCALE * H, UPSCALE * W)


# speedup vs baseline: 452.1428x; 1.0974x over previous
import functools

import jax
import jax.numpy as jnp
from jax import lax
from jax.experimental import pallas as pl
from jax.experimental.pallas import tpu as pltpu
from jax.experimental.pallas import tpu_sc as plsc

L = 16
UPSCALE = 2
H = 2048
W = 2048
WPAD = 2064
NTILES = 32
ROWS_PER_TILE = H // NTILES
CHUNK = 4
NCHUNKS = ROWS_PER_TILE // CHUNK
CBLKS = W // 16
OW = W * UPSCALE

_PERMS = ((0, 1, 2, 3), (2, 0, 3, 1), (3, 2, 1, 0), (1, 3, 0, 2))


def _build_table(lsb_weight):
    wt = 0.5 * jnp.trunc(lsb_weight.astype(jnp.float32))
    parts = []
    for t in range(8):
        k, r = (0, t) if t < 4 else (1, t - 4)
        parts.append(wt[k][:, jnp.array(_PERMS[r])])
    tt = jnp.stack(parts).reshape(8, 16, 16, 4)
    comb = tt[0::2][:, :, :, None, :] + tt[1::2][:, :, None, :, :]
    u = lax.bitcast_convert_type(comb.astype(jnp.bfloat16), jnp.uint16).astype(
        jnp.uint32
    )
    pair01 = u[..., 0] | (u[..., 1] << 16)
    pair23 = u[..., 2] | (u[..., 3] << 16)
    packed = jnp.stack([pair01, pair23], axis=-1)
    return lax.bitcast_convert_type(packed, jnp.int32).reshape(-1)


def _sc_kernel(p_hbm, t_hbm, out_hbm, tbl, ibuf, obuf):
    nc = 2
    wid = lax.axis_index("s") * nc + lax.axis_index("c")
    row0 = wid * ROWS_PER_TILE
    pltpu.sync_copy(t_hbm, tbl)
    ii = lax.iota(jnp.int32, 16)
    ii2 = ii * 2

    def chunk_body(ck, _):
        r_in = row0 + ck * CHUNK
        def stage(d, __):
            pltpu.sync_copy(
                p_hbm.at[pl.ds((r_in + d) * WPAD, WPAD)],
                ibuf.at[pl.ds(d * WPAD, WPAD)],
            )
            return __
        lax.fori_loop(0, CHUNK + 2, stage, 0, unroll=True)

        def px_body(k, __):
            r = k >> 7
            cb = k & 127
            j0 = cb * 16
            off_u = r * WPAD + j0
            off_c = off_u + WPAD
            off_d = off_c + WPAD
            a = ibuf[pl.ds(off_c + 1, 16)]
            nb_r = ibuf[pl.ds(off_c + 2, 16)]
            nb_l = ibuf[pl.ds(off_c, 16)]
            nb_u = ibuf[pl.ds(off_u + 1, 16)]
            nb_d = ibuf[pl.ds(off_d + 1, 16)]
            nb_dr = ibuf[pl.ds(off_d + 2, 16)]
            nb_dl = ibuf[pl.ds(off_d, 16)]
            nb_ul = ibuf[pl.ds(off_u, 16)]
            nb_ur = ibuf[pl.ds(off_u + 2, 16)]
            a512 = a << 9
            acc = [None, None, None, None]
            for q, (n1, n2) in enumerate(
                ((nb_r, nb_d), (nb_l, nb_u), (nb_dr, nb_dl), (nb_ul, nb_ur))
            ):
                base = a512 + (n1 << 5) + (n2 << 1) + (q * 8192)
                g01 = plsc.load_gather(tbl, [base])
                g23 = plsc.load_gather(tbl, [base + 1])
                c0, c1 = plsc.unpack(
                    plsc.bitcast(g01, jnp.bfloat16),
                    format=plsc.PackFormat.INTERLEAVED,
                )
                c2, c3 = plsc.unpack(
                    plsc.bitcast(g23, jnp.bfloat16),
                    format=plsc.PackFormat.INTERLEAVED,
                )
                for c, g in enumerate((c0, c1, c2, c3)):
                    acc[c] = g if acc[c] is None else acc[c] + g
            b00 = (r * 2) * OW + j0 * 2 + ii2
            b10 = b00 + OW
            plsc.store_scatter(obuf, [b00], acc[0])
            plsc.store_scatter(obuf, [b00 + 1], acc[1])
            plsc.store_scatter(obuf, [b10], acc[2])
            plsc.store_scatter(obuf, [b10 + 1], acc[3])
            return __

        lax.fori_loop(0, CHUNK * CBLKS, px_body, 0)
        pltpu.sync_copy(
            obuf, out_hbm.at[pl.ds(r_in * UPSCALE * OW, CHUNK * UPSCALE * OW)]
        )
        return _

    lax.fori_loop(0, NCHUNKS, chunk_body, 0)


@functools.partial(jax.jit, static_argnames=())
def kernel(img_lr, lsb_weight):
    img = img_lr.astype(jnp.int32)
    p = jnp.pad(img, 1, mode="reflect")
    p = jnp.pad(p, ((0, 0), (0, WPAD - (W + 2))))
    p_flat = p.reshape(-1)
    t_flat = _build_table(lsb_weight)

    mesh = plsc.VectorSubcoreMesh(core_axis_name="c", subcore_axis_name="s")
    run = functools.partial(
        pl.kernel,
        mesh=mesh,
        out_type=jax.ShapeDtypeStruct((UPSCALE * H * OW,), jnp.float32),
        scratch_types=[
            pltpu.VMEM((4 * 4096 * 2,), jnp.int32),
            pltpu.VMEM(((CHUNK + 2) * WPAD,), jnp.int32),
            pltpu.VMEM((CHUNK * UPSCALE * OW,), jnp.float32),
        ],
        compiler_params=pltpu.CompilerParams(needs_layout_passes=False),
    )(_sc_kernel)
    out_flat = run(p_flat, t_flat)
    return out_flat.reshape(UPSCALE * H, UPSCALE * W)


# speedup vs baseline: 541.1226x; 1.1968x over previous
import functools

import jax
import jax.numpy as jnp
from jax import lax
from jax.experimental import pallas as pl
from jax.experimental.pallas import tpu as pltpu
from jax.experimental.pallas import tpu_sc as plsc

L = 16
UPSCALE = 2
H = 2048
W = 2048
WPAD = 2064
NTILES = 32
ROWS_PER_TILE = H // NTILES
CHUNK = 4
NCHUNKS = ROWS_PER_TILE // CHUNK
CBLKS = W // 16
OW = W * UPSCALE

_PERMS = ((0, 1, 2, 3), (2, 0, 3, 1), (3, 2, 1, 0), (1, 3, 0, 2))


def _build_table(lsb_weight):
    wt = 0.5 * jnp.trunc(lsb_weight.astype(jnp.float32))
    parts = []
    for t in range(8):
        k, r = (0, t) if t < 4 else (1, t - 4)
        parts.append(wt[k][:, jnp.array(_PERMS[r])])
    tt = jnp.stack(parts).reshape(8, 16, 16, 4)
    comb = tt[0::2][:, :, :, None, :] + tt[1::2][:, :, None, :, :]
    u = lax.bitcast_convert_type(comb.astype(jnp.bfloat16), jnp.uint16).astype(
        jnp.uint32
    )
    pair01 = u[..., 0] | (u[..., 1] << 16)
    pair23 = u[..., 2] | (u[..., 3] << 16)
    packed = jnp.stack([pair01, pair23], axis=-1)
    return lax.bitcast_convert_type(packed, jnp.int32).reshape(-1)


IBSZ = (CHUNK + 2) * WPAD
OBSZ = CHUNK * UPSCALE * OW


def _sc_kernel(p_hbm, t_hbm, out_hbm, tbl, ibuf, obuf, si0, si1, so0, so1):
    nc = 2
    wid = lax.axis_index("s") * nc + lax.axis_index("c")
    row0 = wid * ROWS_PER_TILE
    pltpu.sync_copy(t_hbm, tbl)
    ii = lax.iota(jnp.int32, 16)
    ii2 = ii * 2

    def in_copy(ck, half, sem):
        r_in = row0 + ck * CHUNK
        return pltpu.make_async_copy(
            p_hbm.at[pl.ds(r_in * WPAD, IBSZ)],
            ibuf.at[pl.ds(half * IBSZ, IBSZ)],
            sem,
        )

    def out_copy(ck, half, sem):
        r_in = row0 + ck * CHUNK
        return pltpu.make_async_copy(
            obuf.at[pl.ds(half * OBSZ, OBSZ)],
            out_hbm.at[pl.ds(r_in * UPSCALE * OW, OBSZ)],
            sem,
        )

    def compute_chunk(ib_off, ob_off):
        def px_body(k, __):
            r = k >> 7
            cb = k & 127
            j0 = cb * 16
            off_u = ib_off + r * WPAD + j0
            off_c = off_u + WPAD
            off_d = off_c + WPAD
            a = ibuf[pl.ds(off_c + 1, 16)]
            nb_r = ibuf[pl.ds(off_c + 2, 16)]
            nb_l = ibuf[pl.ds(off_c, 16)]
            nb_u = ibuf[pl.ds(off_u + 1, 16)]
            nb_d = ibuf[pl.ds(off_d + 1, 16)]
            nb_dr = ibuf[pl.ds(off_d + 2, 16)]
            nb_dl = ibuf[pl.ds(off_d, 16)]
            nb_ul = ibuf[pl.ds(off_u, 16)]
            nb_ur = ibuf[pl.ds(off_u + 2, 16)]
            a512 = a << 9
            acc = [None, None, None, None]
            for q, (n1, n2) in enumerate(
                ((nb_r, nb_d), (nb_l, nb_u), (nb_dr, nb_dl), (nb_ul, nb_ur))
            ):
                base = a512 + (n1 << 5) + (n2 << 1) + (q * 8192)
                g01 = plsc.load_gather(tbl, [base])
                g23 = plsc.load_gather(tbl, [base + 1])
                c0, c1 = plsc.unpack(
                    plsc.bitcast(g01, jnp.bfloat16),
                    format=plsc.PackFormat.INTERLEAVED,
                )
                c2, c3 = plsc.unpack(
                    plsc.bitcast(g23, jnp.bfloat16),
                    format=plsc.PackFormat.INTERLEAVED,
                )
                for c, g in enumerate((c0, c1, c2, c3)):
                    acc[c] = g if acc[c] is None else acc[c] + g
            b00 = ob_off + (r * 2) * OW + j0 * 2 + ii2
            b10 = b00 + OW
            plsc.store_scatter(obuf, [b00], acc[0])
            plsc.store_scatter(obuf, [b00 + 1], acc[1])
            plsc.store_scatter(obuf, [b10], acc[2])
            plsc.store_scatter(obuf, [b10 + 1], acc[3])
            return __

        lax.fori_loop(0, CHUNK * CBLKS, px_body, 0)

    in_copy(0, 0, si0).start()
    in_copy(1, 1, si1).start()

    def pair_body(j, _):
        c0 = j * 2
        in_copy(c0, 0, si0).wait()

        @pl.when(j > 0)
        def _w0():
            out_copy(c0 - 2, 0, so0).wait()

        compute_chunk(0, 0)
        out_copy(c0, 0, so0).start()

        @pl.when(j < (NCHUNKS // 2 - 1))
        def _p0():
            in_copy(c0 + 2, 0, si0).start()

        in_copy(c0 + 1, 1, si1).wait()

        @pl.when(j > 0)
        def _w1():
            out_copy(c0 - 1, 1, so1).wait()

        compute_chunk(IBSZ, OBSZ)
        out_copy(c0 + 1, 1, so1).start()

        @pl.when(j < (NCHUNKS // 2 - 1))
        def _p1():
            in_copy(c0 + 3, 1, si1).start()

        return _

    lax.fori_loop(0, NCHUNKS // 2, pair_body, 0)
    out_copy(NCHUNKS - 2, 0, so0).wait()
    out_copy(NCHUNKS - 1, 1, so1).wait()


@functools.partial(jax.jit, static_argnames=())
def kernel(img_lr, lsb_weight):
    img = img_lr.astype(jnp.int32)
    p = jnp.pad(img, 1, mode="reflect")
    p = jnp.pad(p, ((0, 0), (0, WPAD - (W + 2))))
    p_flat = p.reshape(-1)
    t_flat = _build_table(lsb_weight)

    mesh = plsc.VectorSubcoreMesh(core_axis_name="c", subcore_axis_name="s")
    run = functools.partial(
        pl.kernel,
        mesh=mesh,
        out_type=jax.ShapeDtypeStruct((UPSCALE * H * OW,), jnp.float32),
        scratch_types=[
            pltpu.VMEM((4 * 4096 * 2,), jnp.int32),
            pltpu.VMEM((2 * IBSZ,), jnp.int32),
            pltpu.VMEM((2 * OBSZ,), jnp.float32),
            pltpu.SemaphoreType.DMA,
            pltpu.SemaphoreType.DMA,
            pltpu.SemaphoreType.DMA,
            pltpu.SemaphoreType.DMA,
        ],
        compiler_params=pltpu.CompilerParams(needs_layout_passes=False),
    )(_sc_kernel)
    out_flat = run(p_flat, t_flat)
    return out_flat.reshape(UPSCALE * H, UPSCALE * W)


# speedup vs baseline: 560.1061x; 1.0351x over previous
import functools

import jax
import jax.numpy as jnp
from jax import lax
from jax.experimental import pallas as pl
from jax.experimental.pallas import tpu as pltpu
from jax.experimental.pallas import tpu_sc as plsc

L = 16
UPSCALE = 2
H = 2048
W = 2048
WPAD = 2064
NTILES = 32
ROWS_PER_TILE = H // NTILES
CHUNK = 4
NCHUNKS = ROWS_PER_TILE // CHUNK
CBLKS = W // 16
OW = W * UPSCALE

_PERMS = ((0, 1, 2, 3), (2, 0, 3, 1), (3, 2, 1, 0), (1, 3, 0, 2))


def _build_table(lsb_weight):
    wt = 0.5 * jnp.trunc(lsb_weight.astype(jnp.float32))
    parts = []
    for t in range(8):
        k, r = (0, t) if t < 4 else (1, t - 4)
        parts.append(wt[k][:, jnp.array(_PERMS[r])])
    tt = jnp.stack(parts).reshape(8, 16, 16, 4)
    comb = tt[0::2][:, :, :, None, :] + tt[1::2][:, :, None, :, :]
    u = lax.bitcast_convert_type(comb.astype(jnp.bfloat16), jnp.uint16).astype(
        jnp.uint32
    )
    pair01 = u[..., 0] | (u[..., 1] << 16)
    pair23 = u[..., 2] | (u[..., 3] << 16)
    packed = jnp.stack([pair01, pair23], axis=-1)
    packed = packed.transpose(0, 2, 3, 4, 1)
    return lax.bitcast_convert_type(packed, jnp.int32).reshape(-1)


IBSZ = (CHUNK + 2) * WPAD
OBSZ = CHUNK * UPSCALE * OW


def _sc_kernel(p_hbm, t_hbm, out_hbm, tbl, ibuf, obuf, si0, si1, so0, so1):
    nc = 2
    wid = lax.axis_index("s") * nc + lax.axis_index("c")
    row0 = wid * ROWS_PER_TILE
    pltpu.sync_copy(t_hbm, tbl)
    ii = lax.iota(jnp.int32, 16)
    ii2 = ii * 2

    def in_copy(ck, half, sem):
        r_in = row0 + ck * CHUNK
        return pltpu.make_async_copy(
            p_hbm.at[pl.ds(r_in * WPAD, IBSZ)],
            ibuf.at[pl.ds(half * IBSZ, IBSZ)],
            sem,
        )

    def out_copy(ck, half, sem):
        r_in = row0 + ck * CHUNK
        return pltpu.make_async_copy(
            obuf.at[pl.ds(half * OBSZ, OBSZ)],
            out_hbm.at[pl.ds(r_in * UPSCALE * OW, OBSZ)],
            sem,
        )

    def compute_chunk(ib_off, ob_off):
        def px_body(k, __):
            r = k >> 7
            cb = k & 127
            j0 = cb * 16
            off_u = ib_off + r * WPAD + j0
            off_c = off_u + WPAD
            off_d = off_c + WPAD
            a = ibuf[pl.ds(off_c + 1, 16)]
            nb_r = ibuf[pl.ds(off_c + 2, 16)]
            nb_l = ibuf[pl.ds(off_c, 16)]
            nb_u = ibuf[pl.ds(off_u + 1, 16)]
            nb_d = ibuf[pl.ds(off_d + 1, 16)]
            nb_dr = ibuf[pl.ds(off_d + 2, 16)]
            nb_dl = ibuf[pl.ds(off_d, 16)]
            nb_ul = ibuf[pl.ds(off_u, 16)]
            nb_ur = ibuf[pl.ds(off_u + 2, 16)]
            acc = [None, None, None, None]
            for q, (n1, n2) in enumerate(
                ((nb_r, nb_d), (nb_l, nb_u), (nb_dr, nb_dl), (nb_ul, nb_ur))
            ):
                base = a + (n1 << 9) + (n2 << 5) + (q * 8192)
                g01 = plsc.load_gather(tbl, [base])
                g23 = plsc.load_gather(tbl, [base + 16])
                c0, c1 = plsc.unpack(
                    plsc.bitcast(g01, jnp.bfloat16),
                    format=plsc.PackFormat.INTERLEAVED,
                )
                c2, c3 = plsc.unpack(
                    plsc.bitcast(g23, jnp.bfloat16),
                    format=plsc.PackFormat.INTERLEAVED,
                )
                for c, g in enumerate((c0, c1, c2, c3)):
                    acc[c] = g if acc[c] is None else acc[c] + g
            b00 = ob_off + (r * 2) * OW + j0 * 2 + ii2
            b10 = b00 + OW
            plsc.store_scatter(obuf, [b00], acc[0])
            plsc.store_scatter(obuf, [b00 + 1], acc[1])
            plsc.store_scatter(obuf, [b10], acc[2])
            plsc.store_scatter(obuf, [b10 + 1], acc[3])
            return __

        lax.fori_loop(0, CHUNK * CBLKS, px_body, 0)

    in_copy(0, 0, si0).start()
    in_copy(1, 1, si1).start()

    def pair_body(j, _):
        c0 = j * 2
        in_copy(c0, 0, si0).wait()

        @pl.when(j > 0)
        def _w0():
            out_copy(c0 - 2, 0, so0).wait()

        compute_chunk(0, 0)
        out_copy(c0, 0, so0).start()

        @pl.when(j < (NCHUNKS // 2 - 1))
        def _p0():
            in_copy(c0 + 2, 0, si0).start()

        in_copy(c0 + 1, 1, si1).wait()

        @pl.when(j > 0)
        def _w1():
            out_copy(c0 - 1, 1, so1).wait()

        compute_chunk(IBSZ, OBSZ)
        out_copy(c0 + 1, 1, so1).start()

        @pl.when(j < (NCHUNKS // 2 - 1))
        def _p1():
            in_copy(c0 + 3, 1, si1).start()

        return _

    lax.fori_loop(0, NCHUNKS // 2, pair_body, 0)
    out_copy(NCHUNKS - 2, 0, so0).wait()
    out_copy(NCHUNKS - 1, 1, so1).wait()


@functools.partial(jax.jit, static_argnames=())
def kernel(img_lr, lsb_weight):
    img = img_lr.astype(jnp.int32)
    p = jnp.pad(img, 1, mode="reflect")
    p = jnp.pad(p, ((0, 0), (0, WPAD - (W + 2))))
    p_flat = p.reshape(-1)
    t_flat = _build_table(lsb_weight)

    mesh = plsc.VectorSubcoreMesh(core_axis_name="c", subcore_axis_name="s")
    run = functools.partial(
        pl.kernel,
        mesh=mesh,
        out_type=jax.ShapeDtypeStruct((UPSCALE * H * OW,), jnp.float32),
        scratch_types=[
            pltpu.VMEM((4 * 4096 * 2,), jnp.int32),
            pltpu.VMEM((2 * IBSZ,), jnp.int32),
            pltpu.VMEM((2 * OBSZ,), jnp.float32),
            pltpu.SemaphoreType.DMA,
            pltpu.SemaphoreType.DMA,
            pltpu.SemaphoreType.DMA,
            pltpu.SemaphoreType.DMA,
        ],
        compiler_params=pltpu.CompilerParams(needs_layout_passes=False),
    )(_sc_kernel)
    out_flat = run(p_flat, t_flat)
    return out_flat.reshape(UPSCALE * H, UPSCALE * W)


# speedup vs baseline: 892.6659x; 1.5937x over previous
import functools

import jax
import jax.numpy as jnp
from jax import lax
from jax.experimental import pallas as pl
from jax.experimental.pallas import tpu as pltpu
from jax.experimental.pallas import tpu_sc as plsc

L = 16
UPSCALE = 2
H = 2048
W = 2048
NTILES = 32
ROWS_PER_TILE = H // NTILES
CHUNK = 4
NCHUNKS = ROWS_PER_TILE // CHUNK
CBLKS = W // 16
OW = W * UPSCALE
NROW = CHUNK + 2
IBSZ = NROW * W
ORPC = CHUNK * UPSCALE

_PERMS = ((0, 1, 2, 3), (2, 0, 3, 1), (3, 2, 1, 0), (1, 3, 0, 2))


def _build_table(lsb_weight):
    wt = 0.5 * jnp.trunc(lsb_weight.astype(jnp.float32))
    parts = []
    for t in range(8):
        k, r = (0, t) if t < 4 else (1, t - 4)
        parts.append(wt[k][:, jnp.array(_PERMS[r])])
    tt = jnp.stack(parts).reshape(8, 16, 16, 4)
    comb = tt[0::2][:, :, :, None, :] + tt[1::2][:, :, None, :, :]
    u = lax.bitcast_convert_type(comb.astype(jnp.bfloat16), jnp.uint16).astype(
        jnp.uint32
    )
    pair01 = u[..., 0] | (u[..., 1] << 16)
    pair23 = u[..., 2] | (u[..., 3] << 16)
    packed = jnp.stack([pair01, pair23], axis=-1)
    packed = packed.transpose(0, 2, 3, 4, 1)
    return lax.bitcast_convert_type(packed, jnp.int32).reshape(-1)


def _sc_kernel(img_hbm, t_hbm, out_hbm, tbl, ibuf, obuf, si0, si1, so0, so1):
    nc = 2
    wid = lax.axis_index("s") * nc + lax.axis_index("c")
    row0 = wid * ROWS_PER_TILE
    pltpu.sync_copy(t_hbm, tbl)
    ii = lax.iota(jnp.int32, 16)
    ii2 = ii * 2
    zz = ii * 0

    def in_rows(ck, half, sem):
        r_in = row0 + ck * CHUNK
        for d in range(NROW):
            m = r_in + d - 1
            m = jnp.where(m < 0, 1, jnp.where(m > H - 1, H - 2, m))
            yield pltpu.make_async_copy(
                img_hbm.at[m], ibuf.at[pl.ds((half * NROW + d) * W, W)], sem
            )

    def out_copy(ck, half, sem):
        r_in = row0 + ck * CHUNK
        return pltpu.make_async_copy(
            obuf.at[pl.ds(half * ORPC, ORPC), :],
            out_hbm.at[pl.ds(r_in * UPSCALE, ORPC), :],
            sem,
        )

    def compute_chunk(half):
        ib_off = half * IBSZ
        orow0 = half * ORPC

        def px_body(k, __):
            r = k >> 7
            cb = k & 127
            j0 = cb * 16
            jc = ii + j0
            jl = jnp.abs(jc - 1)
            jr = (W - 1) - jnp.abs(jc - (W - 2))
            b_u = ib_off + r * W
            b_c = b_u + W
            b_d = b_c + W
            a = plsc.load_gather(ibuf, [b_c + jc])
            nb_r = plsc.load_gather(ibuf, [b_c + jr])
            nb_l = plsc.load_gather(ibuf, [b_c + jl])
            nb_u = plsc.load_gather(ibuf, [b_u + jc])
            nb_d = plsc.load_gather(ibuf, [b_d + jc])
            nb_dr = plsc.load_gather(ibuf, [b_d + jr])
            nb_dl = plsc.load_gather(ibuf, [b_d + jl])
            nb_ul = plsc.load_gather(ibuf, [b_u + jl])
            nb_ur = plsc.load_gather(ibuf, [b_u + jr])
            acc = [None, None, None, None]
            for q, (n1, n2) in enumerate(
                ((nb_r, nb_d), (nb_l, nb_u), (nb_dr, nb_dl), (nb_ul, nb_ur))
            ):
                base = a + (n1 << 9) + (n2 << 5) + (q * 8192)
                g01 = plsc.load_gather(tbl, [base])
                g23 = plsc.load_gather(tbl, [base + 16])
                c0, c1 = plsc.unpack(
                    plsc.bitcast(g01, jnp.bfloat16),
                    format=plsc.PackFormat.INTERLEAVED,
                )
                c2, c3 = plsc.unpack(
                    plsc.bitcast(g23, jnp.bfloat16),
                    format=plsc.PackFormat.INTERLEAVED,
                )
                for c, g in enumerate((c0, c1, c2, c3)):
                    acc[c] = g if acc[c] is None else acc[c] + g
            row_a = zz + (orow0 + r * 2)
            row_b = row_a + 1
            col0 = ii2 + j0 * 2
            col1 = col0 + 1
            plsc.store_scatter(obuf, [row_a, col0], acc[0])
            plsc.store_scatter(obuf, [row_a, col1], acc[1])
            plsc.store_scatter(obuf, [row_b, col0], acc[2])
            plsc.store_scatter(obuf, [row_b, col1], acc[3])
            return __

        lax.fori_loop(0, CHUNK * CBLKS, px_body, 0)

    for c in in_rows(0, 0, si0):
        c.start()
    for c in in_rows(1, 1, si1):
        c.start()

    def pair_body(j, _):
        c0 = j * 2
        for c in in_rows(c0, 0, si0):
            c.wait()

        @pl.when(j > 0)
        def _w0():
            out_copy(c0 - 2, 0, so0).wait()

        compute_chunk(0)
        out_copy(c0, 0, so0).start()

        @pl.when(j < (NCHUNKS // 2 - 1))
        def _p0():
            for c in in_rows(c0 + 2, 0, si0):
                c.start()

        for c in in_rows(c0 + 1, 1, si1):
            c.wait()

        @pl.when(j > 0)
        def _w1():
            out_copy(c0 - 1, 1, so1).wait()

        compute_chunk(1)
        out_copy(c0 + 1, 1, so1).start()

        @pl.when(j < (NCHUNKS // 2 - 1))
        def _p1():
            for c in in_rows(c0 + 3, 1, si1):
                c.start()

        return _

    lax.fori_loop(0, NCHUNKS // 2, pair_body, 0)
    out_copy(NCHUNKS - 2, 0, so0).wait()
    out_copy(NCHUNKS - 1, 1, so1).wait()


@jax.jit
def kernel(img_lr, lsb_weight):
    img = img_lr.astype(jnp.int32)
    t_flat = _build_table(lsb_weight)

    mesh = plsc.VectorSubcoreMesh(core_axis_name="c", subcore_axis_name="s")
    run = functools.partial(
        pl.kernel,
        mesh=mesh,
        out_type=jax.ShapeDtypeStruct((UPSCALE * H, UPSCALE * W), jnp.float32),
        scratch_types=[
            pltpu.VMEM((4 * 4096 * 2,), jnp.int32),
            pltpu.VMEM((2 * IBSZ,), jnp.int32),
            pltpu.VMEM((2 * ORPC, OW), jnp.float32),
            pltpu.SemaphoreType.DMA,
            pltpu.SemaphoreType.DMA,
            pltpu.SemaphoreType.DMA,
            pltpu.SemaphoreType.DMA,
        ],
        compiler_params=pltpu.CompilerParams(needs_layout_passes=False),
    )(_sc_kernel)
    return run(img, t_flat)


# speedup vs baseline: 1179.3703x; 1.3212x over previous
import functools

import jax
import jax.numpy as jnp
from jax import lax
from jax.experimental import pallas as pl
from jax.experimental.pallas import tpu as pltpu
from jax.experimental.pallas import tpu_sc as plsc

L = 16
UPSCALE = 2
H = 2048
W = 2048
NTILES = 32
ROWS_PER_TILE = H // NTILES
CHUNK = 4
NCHUNKS = ROWS_PER_TILE // CHUNK
CBLKS = W // 16
OW = W * UPSCALE
NROW = CHUNK + 2
IBSZ = NROW * W
ORPC = CHUNK * UPSCALE

_PERMS = ((0, 1, 2, 3), (2, 0, 3, 1), (3, 2, 1, 0), (1, 3, 0, 2))


def _build_table(lsb_weight):
    wt = 0.5 * jnp.trunc(lsb_weight.astype(jnp.float32))
    parts = []
    for t in range(8):
        k, r = (0, t) if t < 4 else (1, t - 4)
        parts.append(wt[k][:, jnp.array(_PERMS[r])])
    tt = jnp.stack(parts).reshape(8, 16, 16, 4)
    comb = tt[0::2][:, :, :, None, :] + tt[1::2][:, :, None, :, :]
    u = lax.bitcast_convert_type(comb.astype(jnp.bfloat16), jnp.uint16).astype(
        jnp.uint32
    )
    pair01 = u[..., 0] | (u[..., 1] << 16)
    pair23 = u[..., 2] | (u[..., 3] << 16)
    packed = jnp.stack([pair01, pair23], axis=-1)
    packed = packed.transpose(0, 2, 3, 4, 1)
    return lax.bitcast_convert_type(packed, jnp.int32).reshape(-1)


def _sc_kernel(img_hbm, t_hbm, out_hbm, tbl, ibuf, obuf, si0, si1, so0, so1):
    nc = 2
    wid = lax.axis_index("s") * nc + lax.axis_index("c")
    row0 = wid * ROWS_PER_TILE
    pltpu.sync_copy(t_hbm, tbl)
    ii = lax.iota(jnp.int32, 16)
    ii2 = ii * 2
    zz = ii * 0

    def in_rows(ck, half, sem):
        r_in = row0 + ck * CHUNK
        for d in range(NROW):
            m = r_in + d - 1
            m = jnp.where(m < 0, 1, jnp.where(m > H - 1, H - 2, m))
            yield pltpu.make_async_copy(
                img_hbm.at[m], ibuf.at[pl.ds((half * NROW + d) * W, W)], sem
            )

    def out_copy(ck, half, sem):
        r_in = row0 + ck * CHUNK
        return pltpu.make_async_copy(
            obuf.at[pl.ds(half * ORPC, ORPC), :],
            out_hbm.at[pl.ds(r_in * UPSCALE, ORPC), :],
            sem,
        )

    def compute_chunk(half):
        ib_off = half * IBSZ
        obase = [zz + (half * ORPC + rr) for rr in range(ORPC)]

        def cb_body(cb, __):
            j0 = cb * 16
            jc = ii + j0
            jl = jnp.abs(jc - 1)
            jr = (W - 1) - jnp.abs(jc - (W - 2))
            rows = []
            for d in range(NROW):
                b = ib_off + d * W
                rows.append((
                    plsc.load_gather(ibuf, [b + jl]),
                    plsc.load_gather(ibuf, [b + jc]),
                    plsc.load_gather(ibuf, [b + jr]),
                ))
            col0 = ii2 + j0 * 2
            col1 = col0 + 1
            shifted = {}

            def shift(key, v, s):
                k = (key, s)
                if k not in shifted:
                    shifted[k] = v << s
                return shifted[k]

            for r in range(CHUNK):
                upl, upc, upr = rows[r]
                cel, a, cer = rows[r + 1]
                dnl, dnc, dnr = rows[r + 2]
                qs = (
                    ((("r", r + 1), cer), (("c", r + 2), dnc)),
                    ((("l", r + 1), cel), (("c", r), upc)),
                    ((("r", r + 2), dnr), (("l", r + 2), dnl)),
                    ((("l", r), upl), (("r", r), upr)),
                )
                p01 = []
                p23 = []
                for q, ((k1, v1), (k2, v2)) in enumerate(qs):
                    base = a + shift(k1, v1, 9) + shift(k2, v2, 5) + q * 8192
                    g01 = plsc.load_gather(tbl, [base])
                    g23 = plsc.load_gather(tbl, [base + 16])
                    p01.append(plsc.bitcast(g01, jnp.bfloat16))
                    p23.append(plsc.bitcast(g23, jnp.bfloat16))
                ipk = plsc.PackFormat.INTERLEAVED
                c0a, c1a = plsc.unpack(p01[0] + p01[1], format=ipk)
                c0b, c1b = plsc.unpack(p01[2] + p01[3], format=ipk)
                c2a, c3a = plsc.unpack(p23[0] + p23[1], format=ipk)
                c2b, c3b = plsc.unpack(p23[2] + p23[3], format=ipk)
                ra = obase[2 * r]
                rb = obase[2 * r + 1]
                plsc.store_scatter(obuf, [ra, col0], c0a + c0b)
                plsc.store_scatter(obuf, [ra, col1], c1a + c1b)
                plsc.store_scatter(obuf, [rb, col0], c2a + c2b)
                plsc.store_scatter(obuf, [rb, col1], c3a + c3b)
            return __

        lax.fori_loop(0, CBLKS, cb_body, 0)

    for c in in_rows(0, 0, si0):
        c.start()
    for c in in_rows(1, 1, si1):
        c.start()

    def pair_body(j, _):
        c0 = j * 2
        for c in in_rows(c0, 0, si0):
            c.wait()

        @pl.when(j > 0)
        def _w0():
            out_copy(c0 - 2, 0, so0).wait()

        compute_chunk(0)
        out_copy(c0, 0, so0).start()

        @pl.when(j < (NCHUNKS // 2 - 1))
        def _p0():
            for c in in_rows(c0 + 2, 0, si0):
                c.start()

        for c in in_rows(c0 + 1, 1, si1):
            c.wait()

        @pl.when(j > 0)
        def _w1():
            out_copy(c0 - 1, 1, so1).wait()

        compute_chunk(1)
        out_copy(c0 + 1, 1, so1).start()

        @pl.when(j < (NCHUNKS // 2 - 1))
        def _p1():
            for c in in_rows(c0 + 3, 1, si1):
                c.start()

        return _

    lax.fori_loop(0, NCHUNKS // 2, pair_body, 0)
    out_copy(NCHUNKS - 2, 0, so0).wait()
    out_copy(NCHUNKS - 1, 1, so1).wait()


@jax.jit
def kernel(img_lr, lsb_weight):
    img = img_lr.astype(jnp.int32)
    t_flat = _build_table(lsb_weight)

    mesh = plsc.VectorSubcoreMesh(core_axis_name="c", subcore_axis_name="s")
    run = functools.partial(
        pl.kernel,
        mesh=mesh,
        out_type=jax.ShapeDtypeStruct((UPSCALE * H, UPSCALE * W), jnp.float32),
        scratch_types=[
            pltpu.VMEM((4 * 4096 * 2,), jnp.int32),
            pltpu.VMEM((2 * IBSZ,), jnp.int32),
            pltpu.VMEM((2 * ORPC, OW), jnp.float32),
            pltpu.SemaphoreType.DMA,
            pltpu.SemaphoreType.DMA,
            pltpu.SemaphoreType.DMA,
            pltpu.SemaphoreType.DMA,
        ],
        compiler_params=pltpu.CompilerParams(needs_layout_passes=False),
    )(_sc_kernel)
    return run(img, t_flat)


# speedup vs baseline: 1287.9585x; 1.0921x over previous
import functools

import jax
import jax.numpy as jnp
from jax import lax
from jax.experimental import pallas as pl
from jax.experimental.pallas import tpu as pltpu
from jax.experimental.pallas import tpu_sc as plsc

L = 16
UPSCALE = 2
H = 2048
W = 2048
NTILES = 32
ROWS_PER_TILE = H // NTILES
CHUNK = 4
NCHUNKS = ROWS_PER_TILE // CHUNK
CBLKS = W // 16
OW = W * UPSCALE
NROW = CHUNK + 2
WROW = 2304
IMG0 = 128
IBSZ = NROW * WROW
ORPC = CHUNK * UPSCALE

_PERMS = ((0, 1, 2, 3), (2, 0, 3, 1), (3, 2, 1, 0), (1, 3, 0, 2))


def _build_table(lsb_weight):
    wt = 0.5 * jnp.trunc(lsb_weight.astype(jnp.float32))
    parts = []
    for t in range(8):
        k, r = (0, t) if t < 4 else (1, t - 4)
        parts.append(wt[k][:, jnp.array(_PERMS[r])])
    tt = jnp.stack(parts).reshape(8, 16, 16, 4)
    comb = tt[0::2][:, :, :, None, :] + tt[1::2][:, :, None, :, :]
    u = lax.bitcast_convert_type(comb.astype(jnp.bfloat16), jnp.uint16).astype(
        jnp.uint32
    )
    pair01 = u[..., 0] | (u[..., 1] << 16)
    pair23 = u[..., 2] | (u[..., 3] << 16)
    packed = jnp.stack([pair01, pair23], axis=-1)
    packed = packed.transpose(0, 2, 3, 4, 1)
    return lax.bitcast_convert_type(packed, jnp.int32).reshape(-1)


def _sc_kernel(img_hbm, t_hbm, out_hbm, tbl, ibuf, obuf, si0, si1, so0, so1):
    nc = 2
    wid = lax.axis_index("s") * nc + lax.axis_index("c")
    row0 = wid * ROWS_PER_TILE
    pltpu.sync_copy(t_hbm, tbl)
    ii = lax.iota(jnp.int32, 16)
    ii2 = ii * 2
    zz = ii * 0

    def in_rows(ck, half, sem):
        r_in = row0 + ck * CHUNK
        for d in range(NROW):
            m = r_in + d - 1
            m = jnp.where(m < 0, 1, jnp.where(m > H - 1, H - 2, m))
            yield pltpu.make_async_copy(
                img_hbm.at[m],
                ibuf.at[pl.ds((half * NROW + d) * WROW + IMG0, W)],
                sem,
            )

    gmask = ii < 2 * NROW
    gd = jnp.minimum(ii >> 1, NROW - 1)
    gs = ii & 1
    gsrc0 = gd * WROW + (IMG0 + 1) + gs * (W - 3)

    def fix_guards(half):
        src = gsrc0 + half * IBSZ
        vals = plsc.load_gather(ibuf, [src], mask=gmask)
        plsc.store_scatter(ibuf, [src - 2 + gs * 4], vals, mask=gmask)

    def out_copy(ck, half, sem):
        r_in = row0 + ck * CHUNK
        return pltpu.make_async_copy(
            obuf.at[pl.ds(half * ORPC, ORPC), :],
            out_hbm.at[pl.ds(r_in * UPSCALE, ORPC), :],
            sem,
        )

    def compute_chunk(half):
        ib_off = half * IBSZ
        obase = [zz + (half * ORPC + rr) for rr in range(ORPC)]

        def cb_body(cb, __):
            j0 = cb * 16
            rows = []
            for d in range(NROW):
                b = ib_off + d * WROW + IMG0 + j0
                rows.append((
                    ibuf[pl.ds(b - 1, 16)],
                    ibuf[pl.ds(b, 16)],
                    ibuf[pl.ds(b + 1, 16)],
                ))
            col0 = ii2 + j0 * 2
            col1 = col0 + 1
            shifted = {}

            def shift(key, v, s):
                k = (key, s)
                if k not in shifted:
                    shifted[k] = v << s
                return shifted[k]

            for r in range(CHUNK):
                upl, upc, upr = rows[r]
                cel, a, cer = rows[r + 1]
                dnl, dnc, dnr = rows[r + 2]
                qs = (
                    ((("r", r + 1), cer), (("c", r + 2), dnc)),
                    ((("l", r + 1), cel), (("c", r), upc)),
                    ((("r", r + 2), dnr), (("l", r + 2), dnl)),
                    ((("l", r), upl), (("r", r), upr)),
                )
                p01 = []
                p23 = []
                for q, ((k1, v1), (k2, v2)) in enumerate(qs):
                    base = a + shift(k1, v1, 9) + shift(k2, v2, 5) + q * 8192
                    g01 = plsc.load_gather(tbl, [base])
                    g23 = plsc.load_gather(tbl, [base + 16])
                    p01.append(plsc.bitcast(g01, jnp.bfloat16))
                    p23.append(plsc.bitcast(g23, jnp.bfloat16))
                ipk = plsc.PackFormat.INTERLEAVED
                acc01 = ((p01[0] + p01[1]) + p01[2]) + p01[3]
                acc23 = ((p23[0] + p23[1]) + p23[2]) + p23[3]
                c0, c1 = plsc.unpack(acc01, format=ipk)
                c2, c3 = plsc.unpack(acc23, format=ipk)
                ra = obase[2 * r]
                rb = obase[2 * r + 1]
                plsc.store_scatter(obuf, [ra, col0], c0)
                plsc.store_scatter(obuf, [ra, col1], c1)
                plsc.store_scatter(obuf, [rb, col0], c2)
                plsc.store_scatter(obuf, [rb, col1], c3)
            return __

        lax.fori_loop(0, CBLKS, cb_body, 0)

    for c in in_rows(0, 0, si0):
        c.start()
    for c in in_rows(1, 1, si1):
        c.start()

    def pair_body(j, _):
        c0 = j * 2
        for c in in_rows(c0, 0, si0):
            c.wait()
        fix_guards(0)

        @pl.when(j > 0)
        def _w0():
            out_copy(c0 - 2, 0, so0).wait()

        compute_chunk(0)
        out_copy(c0, 0, so0).start()

        @pl.when(j < (NCHUNKS // 2 - 1))
        def _p0():
            for c in in_rows(c0 + 2, 0, si0):
                c.start()

        for c in in_rows(c0 + 1, 1, si1):
            c.wait()
        fix_guards(1)

        @pl.when(j > 0)
        def _w1():
            out_copy(c0 - 1, 1, so1).wait()

        compute_chunk(1)
        out_copy(c0 + 1, 1, so1).start()

        @pl.when(j < (NCHUNKS // 2 - 1))
        def _p1():
            for c in in_rows(c0 + 3, 1, si1):
                c.start()

        return _

    lax.fori_loop(0, NCHUNKS // 2, pair_body, 0)
    out_copy(NCHUNKS - 2, 0, so0).wait()
    out_copy(NCHUNKS - 1, 1, so1).wait()


@jax.jit
def kernel(img_lr, lsb_weight):
    img = img_lr.astype(jnp.int32)
    t_flat = _build_table(lsb_weight)

    mesh = plsc.VectorSubcoreMesh(core_axis_name="c", subcore_axis_name="s")
    run = functools.partial(
        pl.kernel,
        mesh=mesh,
        out_type=jax.ShapeDtypeStruct((UPSCALE * H, UPSCALE * W), jnp.float32),
        scratch_types=[
            pltpu.VMEM((4 * 4096 * 2,), jnp.int32),
            pltpu.VMEM((2 * IBSZ,), jnp.int32),
            pltpu.VMEM((2 * ORPC, OW), jnp.float32),
            pltpu.SemaphoreType.DMA,
            pltpu.SemaphoreType.DMA,
            pltpu.SemaphoreType.DMA,
            pltpu.SemaphoreType.DMA,
        ],
        compiler_params=pltpu.CompilerParams(needs_layout_passes=False),
    )(_sc_kernel)
    return run(img, t_flat)


# speedup vs baseline: 1310.8177x; 1.0177x over previous
import functools

import jax
import jax.numpy as jnp
from jax import lax
from jax.experimental import pallas as pl
from jax.experimental.pallas import tpu as pltpu
from jax.experimental.pallas import tpu_sc as plsc

L = 16
UPSCALE = 2
H = 2048
W = 2048
NTILES = 32
ROWS_PER_TILE = H // NTILES
CHUNK = 4
NCHUNKS = ROWS_PER_TILE // CHUNK
CBLKS = W // 16
OW = W * UPSCALE
NROW = CHUNK + 2
WROW = 2304
IMG0 = 128
IBSZ = NROW * WROW
ORPC = CHUNK * UPSCALE

_PERMS = ((0, 1, 2, 3), (2, 0, 3, 1), (3, 2, 1, 0), (1, 3, 0, 2))


def _build_table(lsb_weight):
    wt = 0.5 * jnp.trunc(lsb_weight.astype(jnp.float32))
    parts = []
    for t in range(8):
        k, r = (0, t) if t < 4 else (1, t - 4)
        parts.append(wt[k][:, jnp.array(_PERMS[r])])
    tt = jnp.stack(parts).reshape(8, 16, 16, 4)
    comb = tt[0::2][:, :, :, None, :] + tt[1::2][:, :, None, :, :]
    u = lax.bitcast_convert_type(comb.astype(jnp.bfloat16), jnp.uint16).astype(
        jnp.uint32
    )
    pair01 = u[..., 0] | (u[..., 1] << 16)
    pair23 = u[..., 2] | (u[..., 3] << 16)
    packed = jnp.stack([pair01, pair23], axis=-1)
    packed = packed.transpose(0, 2, 3, 4, 1)
    return lax.bitcast_convert_type(packed, jnp.int32).reshape(-1)


def _sc_kernel(img_hbm, t_hbm, out_hbm, tbl, ibuf, obuf, si0, si1, so0, so1):
    nc = 2
    wid = lax.axis_index("s") * nc + lax.axis_index("c")
    row0 = wid * ROWS_PER_TILE
    pltpu.sync_copy(t_hbm, tbl)
    ii = lax.iota(jnp.int32, 16)
    ii2 = ii * 2
    zz = ii * 0

    def in_rows(ck, half, sem):
        r_in = row0 + ck * CHUNK
        for d in range(NROW):
            m = r_in + d - 1
            m = jnp.where(m < 0, 1, jnp.where(m > H - 1, H - 2, m))
            yield pltpu.make_async_copy(
                img_hbm.at[m],
                ibuf.at[pl.ds((half * NROW + d) * WROW + IMG0, W)],
                sem,
            )

    gmask = ii < 2 * NROW
    gd = jnp.minimum(ii >> 1, NROW - 1)
    gs = ii & 1
    gsrc0 = gd * WROW + (IMG0 + 1) + gs * (W - 3)

    def fix_guards(half):
        src = gsrc0 + half * IBSZ
        vals = plsc.load_gather(ibuf, [src], mask=gmask)
        plsc.store_scatter(ibuf, [src - 2 + gs * 4], vals, mask=gmask)

    def out_copy(ck, half, sem):
        r_in = row0 + ck * CHUNK
        return pltpu.make_async_copy(
            obuf.at[pl.ds(half * ORPC, ORPC), :],
            out_hbm.at[pl.ds(r_in * UPSCALE, ORPC), :],
            sem,
        )

    tbl01 = [tbl.at[pl.ds(q * 8192, 8192)] for q in range(4)]
    tbl23 = [tbl.at[pl.ds(q * 8192 + 16, 8176)] for q in range(4)]

    def compute_chunk(half):
        ib_off = half * IBSZ
        obase = [zz + (half * ORPC + rr) for rr in range(ORPC)]

        def cb_body(cb, __):
            j0 = cb * 16
            rows = []
            for d in range(NROW):
                b = ib_off + d * WROW + IMG0 + j0
                rows.append((
                    ibuf[pl.ds(b - 1, 16)],
                    ibuf[pl.ds(b, 16)],
                    ibuf[pl.ds(b + 1, 16)],
                ))
            col0 = ii2 + j0 * 2
            col1 = col0 + 1
            shifted = {}

            def shift(key, v, s):
                k = (key, s)
                if k not in shifted:
                    shifted[k] = v << s
                return shifted[k]

            for r in range(CHUNK):
                upl, upc, upr = rows[r]
                cel, a, cer = rows[r + 1]
                dnl, dnc, dnr = rows[r + 2]
                qs = (
                    ((("r", r + 1), cer), (("c", r + 2), dnc)),
                    ((("l", r + 1), cel), (("c", r), upc)),
                    ((("r", r + 2), dnr), (("l", r + 2), dnl)),
                    ((("l", r), upl), (("r", r), upr)),
                )
                p01 = []
                p23 = []
                for q, ((k1, v1), (k2, v2)) in enumerate(qs):
                    base = a + shift(k1, v1, 9) + shift(k2, v2, 5)
                    g01 = plsc.load_gather(tbl01[q], [base])
                    g23 = plsc.load_gather(tbl23[q], [base])
                    p01.append(plsc.bitcast(g01, jnp.bfloat16))
                    p23.append(plsc.bitcast(g23, jnp.bfloat16))
                ipk = plsc.PackFormat.INTERLEAVED
                acc01 = ((p01[0] + p01[1]) + p01[2]) + p01[3]
                acc23 = ((p23[0] + p23[1]) + p23[2]) + p23[3]
                c0, c1 = plsc.unpack(acc01, format=ipk)
                c2, c3 = plsc.unpack(acc23, format=ipk)
                ra = obase[2 * r]
                rb = obase[2 * r + 1]
                plsc.store_scatter(obuf, [ra, col0], c0)
                plsc.store_scatter(obuf, [ra, col1], c1)
                plsc.store_scatter(obuf, [rb, col0], c2)
                plsc.store_scatter(obuf, [rb, col1], c3)
            return __

        lax.fori_loop(0, CBLKS, cb_body, 0)

    for c in in_rows(0, 0, si0):
        c.start()
    for c in in_rows(1, 1, si1):
        c.start()

    def pair_body(j, _):
        c0 = j * 2
        for c in in_rows(c0, 0, si0):
            c.wait()
        fix_guards(0)

        @pl.when(j > 0)
        def _w0():
            out_copy(c0 - 2, 0, so0).wait()

        compute_chunk(0)
        out_copy(c0, 0, so0).start()

        @pl.when(j < (NCHUNKS // 2 - 1))
        def _p0():
            for c in in_rows(c0 + 2, 0, si0):
                c.start()

        for c in in_rows(c0 + 1, 1, si1):
            c.wait()
        fix_guards(1)

        @pl.when(j > 0)
        def _w1():
            out_copy(c0 - 1, 1, so1).wait()

        compute_chunk(1)
        out_copy(c0 + 1, 1, so1).start()

        @pl.when(j < (NCHUNKS // 2 - 1))
        def _p1():
            for c in in_rows(c0 + 3, 1, si1):
                c.start()

        return _

    lax.fori_loop(0, NCHUNKS // 2, pair_body, 0)
    out_copy(NCHUNKS - 2, 0, so0).wait()
    out_copy(NCHUNKS - 1, 1, so1).wait()


@jax.jit
def kernel(img_lr, lsb_weight):
    img = img_lr.astype(jnp.int32)
    t_flat = _build_table(lsb_weight)

    mesh = plsc.VectorSubcoreMesh(core_axis_name="c", subcore_axis_name="s")
    run = functools.partial(
        pl.kernel,
        mesh=mesh,
        out_type=jax.ShapeDtypeStruct((UPSCALE * H, UPSCALE * W), jnp.float32),
        scratch_types=[
            pltpu.VMEM((4 * 4096 * 2,), jnp.int32),
            pltpu.VMEM((2 * IBSZ,), jnp.int32),
            pltpu.VMEM((2 * ORPC, OW), jnp.float32),
            pltpu.SemaphoreType.DMA,
            pltpu.SemaphoreType.DMA,
            pltpu.SemaphoreType.DMA,
            pltpu.SemaphoreType.DMA,
        ],
        compiler_params=pltpu.CompilerParams(needs_layout_passes=False),
    )(_sc_kernel)
    return run(img, t_flat)
